# baseline (speedup 1.0000x reference)
import jax
import jax.numpy as jnp
from jax import lax
from jax.experimental import pallas as pl
from jax.experimental.pallas import tpu as pltpu
from jax.experimental.pallas import tpu_sc as plsc

_FMIN = 0.1
_FMAX = 0.5

_B, _T, _F = 64, 512, 1024
_ROWS = _B * _T
_NW = 32
_RPW = _ROWS // _NW
_G = 32
_STEPS = _RPW // _G
_L = 16


def _row_argmax(buf, r):
    lanes = lax.iota(jnp.int32, _L)

    def chunk(c, carry):
        best, besti, ivec = carry
        v = buf[r, pl.ds(c * _L, _L)]
        m = v > best
        best = jnp.where(m, v, best)
        besti = jnp.where(m, ivec, besti)
        return best, besti, ivec + _L

    best0 = jnp.full((_L,), -jnp.inf, jnp.float32)
    best, besti, _ = lax.fori_loop(
        0, _F // _L, chunk, (best0, lanes, lanes), unroll=16
    )
    return best, besti


def _perm(x, pidx):
    return lax.gather(
        x,
        pidx[:, None],
        lax.GatherDimensionNumbers(
            offset_dims=(), collapsed_slice_dims=(0,), start_index_map=(0,)
        ),
        slice_sizes=(1,),
        mode=lax.GatherScatterMode.PROMISE_IN_BOUNDS,
    )


def _lane_argmax(best, besti):
    lanes = lax.iota(jnp.int32, _L)
    for shift in (8, 4, 2, 1):
        pidx = lanes ^ shift
        pv = _perm(best, pidx)
        pi = _perm(besti, pidx)
        take = (pv > best) | ((pv == best) & (pi < besti))
        best = jnp.where(take, pv, best)
        besti = jnp.where(take, pi, besti)
    return besti


def _compute_block(buf, out_v, step):
    lanes = lax.iota(jnp.int32, _L)
    scale = (_FMAX - _FMIN) / (_F - 1) * 60.0
    bias = _FMIN * 60.0

    def group(g, _):
        def row(r16, acc):
            best, besti = _row_argmax(buf, g * _L + r16)
            wini = _lane_argmax(best, besti)
            rr = wini.astype(jnp.float32) * scale + bias
            return jnp.where(lanes == r16, rr, acc)

        acc = lax.fori_loop(0, _L, row, jnp.zeros((_L,), jnp.float32))
        out_v[pl.ds(step * _G + g * _L, _L)] = acc
        return 0

    lax.fori_loop(0, _G // _L, group, 0)


def _psd_body(x_hbm, out_hbm, buf0, buf1, out_v, sem0, sem1):
    wid = lax.axis_index("s") * 2 + lax.axis_index("c")
    row0 = wid * _RPW

    def start(step, buf, sem):
        pltpu.async_copy(x_hbm.at[pl.ds(row0 + step * _G, _G)], buf, sem)

    def wait(step, buf, sem):
        pltpu.make_async_copy(x_hbm.at[pl.ds(row0 + step * _G, _G)], buf, sem).wait()

    start(0, buf0, sem0)

    def pair(p, _):
        s0 = p * 2
        wait(s0, buf0, sem0)
        start(s0 + 1, buf1, sem1)
        _compute_block(buf0, out_v, s0)
        wait(s0 + 1, buf1, sem1)

        @pl.when(p < _STEPS // 2 - 1)
        def _():
            start(s0 + 2, buf0, sem0)

        _compute_block(buf1, out_v, s0 + 1)
        return 0

    lax.fori_loop(0, _STEPS // 2, pair, 0)
    pltpu.sync_copy(out_v, out_hbm.at[pl.ds(row0, _RPW)])


@jax.jit
def _psd_peaks(x2d):
    mesh = plsc.VectorSubcoreMesh(
        core_axis_name="c", subcore_axis_name="s", num_cores=2, num_subcores=16
    )
    return pl.kernel(
        _psd_body,
        out_type=jax.ShapeDtypeStruct((_ROWS,), jnp.float32),
        mesh=mesh,
        scratch_types=[
            pltpu.VMEM((_G, _F), jnp.float32),
            pltpu.VMEM((_G, _F), jnp.float32),
            pltpu.VMEM((_RPW,), jnp.float32),
            pltpu.SemaphoreType.DMA,
            pltpu.SemaphoreType.DMA,
        ],
    )(x2d)


def kernel(x):
    rr = _psd_peaks(x.reshape(_ROWS, _F))
    return rr.reshape(_B, _T)


# speedup vs baseline: 1.4326x; 1.4326x over previous
import jax
import jax.numpy as jnp
from jax import lax
from jax.experimental import pallas as pl
from jax.experimental.pallas import tpu as pltpu
from jax.experimental.pallas import tpu_sc as plsc

_FMIN = 0.1
_FMAX = 0.5

_B, _T, _F = 64, 512, 1024
_ROWS = _B * _T
_NW = 32
_RPW = _ROWS // _NW
_G = 32
_STEPS = _RPW // _G
_L = 16


_RPAR = 4


def _rows_argmax(buf, rbase):
    lanes = lax.iota(jnp.int32, _L)

    def chunk(c, carry):
        bests, bestis, ivec = carry
        nb, ni = [], []
        for k in range(_RPAR):
            v = buf[rbase + k, pl.ds(c * _L, _L)]
            m = v > bests[k]
            nb.append(jnp.maximum(bests[k], v))
            ni.append(jnp.where(m, ivec, bestis[k]))
        return tuple(nb), tuple(ni), ivec + _L

    best0 = jnp.full((_L,), -jnp.inf, jnp.float32)
    bests, bestis, _ = lax.fori_loop(
        0, _F // _L, chunk,
        ((best0,) * _RPAR, (lanes,) * _RPAR, lanes),
        unroll=8,
    )
    return bests, bestis


def _perm(x, pidx):
    return lax.gather(
        x,
        pidx[:, None],
        lax.GatherDimensionNumbers(
            offset_dims=(), collapsed_slice_dims=(0,), start_index_map=(0,)
        ),
        slice_sizes=(1,),
        mode=lax.GatherScatterMode.PROMISE_IN_BOUNDS,
    )


def _lane_argmax(best, besti):
    lanes = lax.iota(jnp.int32, _L)
    for shift in (8, 4, 2, 1):
        pidx = lanes ^ shift
        pv = _perm(best, pidx)
        pi = _perm(besti, pidx)
        take = (pv > best) | ((pv == best) & (pi < besti))
        best = jnp.where(take, pv, best)
        besti = jnp.where(take, pi, besti)
    return besti


def _compute_block(buf, out_v, step):
    lanes = lax.iota(jnp.int32, _L)
    scale = (_FMAX - _FMIN) / (_F - 1) * 60.0
    bias = _FMIN * 60.0

    def group(g, _):
        def quad(r4, acc):
            bests, bestis = _rows_argmax(buf, g * _L + r4 * _RPAR)
            for k in range(_RPAR):
                wini = _lane_argmax(bests[k], bestis[k])
                rr = wini.astype(jnp.float32) * scale + bias
                acc = jnp.where(lanes == r4 * _RPAR + k, rr, acc)
            return acc

        acc = lax.fori_loop(0, _L // _RPAR, quad, jnp.zeros((_L,), jnp.float32))
        out_v[pl.ds(step * _G + g * _L, _L)] = acc
        return 0

    lax.fori_loop(0, _G // _L, group, 0)


def _psd_body(x_hbm, out_hbm, buf0, buf1, out_v, sem0, sem1):
    wid = lax.axis_index("s") * 2 + lax.axis_index("c")
    row0 = wid * _RPW

    def start(step, buf, sem):
        pltpu.async_copy(x_hbm.at[pl.ds(row0 + step * _G, _G)], buf, sem)

    def wait(step, buf, sem):
        pltpu.make_async_copy(x_hbm.at[pl.ds(row0 + step * _G, _G)], buf, sem).wait()

    start(0, buf0, sem0)

    def pair(p, _):
        s0 = p * 2
        wait(s0, buf0, sem0)
        start(s0 + 1, buf1, sem1)
        _compute_block(buf0, out_v, s0)
        wait(s0 + 1, buf1, sem1)

        @pl.when(p < _STEPS // 2 - 1)
        def _():
            start(s0 + 2, buf0, sem0)

        _compute_block(buf1, out_v, s0 + 1)
        return 0

    lax.fori_loop(0, _STEPS // 2, pair, 0)
    pltpu.sync_copy(out_v, out_hbm.at[pl.ds(row0, _RPW)])


@jax.jit
def _psd_peaks(x2d):
    mesh = plsc.VectorSubcoreMesh(
        core_axis_name="c", subcore_axis_name="s", num_cores=2, num_subcores=16
    )
    return pl.kernel(
        _psd_body,
        out_type=jax.ShapeDtypeStruct((_ROWS,), jnp.float32),
        mesh=mesh,
        scratch_types=[
            pltpu.VMEM((_G, _F), jnp.float32),
            pltpu.VMEM((_G, _F), jnp.float32),
            pltpu.VMEM((_RPW,), jnp.float32),
            pltpu.SemaphoreType.DMA,
            pltpu.SemaphoreType.DMA,
        ],
    )(x2d)


def kernel(x):
    rr = _psd_peaks(x.reshape(_ROWS, _F))
    return rr.reshape(_B, _T)


# speedup vs baseline: 1.4332x; 1.0004x over previous
import jax
import jax.numpy as jnp
from jax import lax
from jax.experimental import pallas as pl
from jax.experimental.pallas import tpu as pltpu
from jax.experimental.pallas import tpu_sc as plsc

_FMIN = 0.1
_FMAX = 0.5

_B, _T, _F = 64, 512, 1024
_ROWS = _B * _T
_NW = 32
_RPW = _ROWS // _NW
_G = 32
_STEPS = _RPW // _G
_L = 16


_RPAR = 8


def _rows_argmax(buf, rbase):
    lanes = lax.iota(jnp.int32, _L)

    def chunk(c, carry):
        bests, bestis, ivec = carry
        nb, ni = [], []
        for k in range(_RPAR):
            v = buf[rbase + k, pl.ds(c * _L, _L)]
            m = v > bests[k]
            nb.append(jnp.maximum(bests[k], v))
            ni.append(jnp.where(m, ivec, bestis[k]))
        return tuple(nb), tuple(ni), ivec + _L

    best0 = jnp.full((_L,), -jnp.inf, jnp.float32)
    bests, bestis, _ = lax.fori_loop(
        0, _F // _L, chunk,
        ((best0,) * _RPAR, (lanes,) * _RPAR, lanes),
        unroll=4,
    )
    return bests, bestis


def _perm(x, pidx):
    return lax.gather(
        x,
        pidx[:, None],
        lax.GatherDimensionNumbers(
            offset_dims=(), collapsed_slice_dims=(0,), start_index_map=(0,)
        ),
        slice_sizes=(1,),
        mode=lax.GatherScatterMode.PROMISE_IN_BOUNDS,
    )


def _lane_argmax(best, besti):
    lanes = lax.iota(jnp.int32, _L)
    for shift in (8, 4, 2, 1):
        pidx = lanes ^ shift
        pv = _perm(best, pidx)
        pi = _perm(besti, pidx)
        take = (pv > best) | ((pv == best) & (pi < besti))
        best = jnp.where(take, pv, best)
        besti = jnp.where(take, pi, besti)
    return besti


def _compute_block(buf, out_v, step):
    lanes = lax.iota(jnp.int32, _L)
    scale = (_FMAX - _FMIN) / (_F - 1) * 60.0
    bias = _FMIN * 60.0

    def group(g, _):
        def quad(r4, acc):
            bests, bestis = _rows_argmax(buf, g * _L + r4 * _RPAR)
            for k in range(_RPAR):
                wini = _lane_argmax(bests[k], bestis[k])
                rr = wini.astype(jnp.float32) * scale + bias
                acc = jnp.where(lanes == r4 * _RPAR + k, rr, acc)
            return acc

        acc = lax.fori_loop(0, _L // _RPAR, quad, jnp.zeros((_L,), jnp.float32))
        out_v[pl.ds(step * _G + g * _L, _L)] = acc
        return 0

    lax.fori_loop(0, _G // _L, group, 0)


def _psd_body(x_hbm, out_hbm, buf0, buf1, out_v, sem0, sem1):
    wid = lax.axis_index("s") * 2 + lax.axis_index("c")
    row0 = wid * _RPW

    def start(step, buf, sem):
        pltpu.async_copy(x_hbm.at[pl.ds(row0 + step * _G, _G)], buf, sem)

    def wait(step, buf, sem):
        pltpu.make_async_copy(x_hbm.at[pl.ds(row0 + step * _G, _G)], buf, sem).wait()

    start(0, buf0, sem0)

    def pair(p, _):
        s0 = p * 2
        wait(s0, buf0, sem0)
        start(s0 + 1, buf1, sem1)
        _compute_block(buf0, out_v, s0)
        wait(s0 + 1, buf1, sem1)

        @pl.when(p < _STEPS // 2 - 1)
        def _():
            start(s0 + 2, buf0, sem0)

        _compute_block(buf1, out_v, s0 + 1)
        return 0

    lax.fori_loop(0, _STEPS // 2, pair, 0)
    pltpu.sync_copy(out_v, out_hbm.at[pl.ds(row0, _RPW)])


@jax.jit
def _psd_peaks(x2d):
    mesh = plsc.VectorSubcoreMesh(
        core_axis_name="c", subcore_axis_name="s", num_cores=2, num_subcores=16
    )
    return pl.kernel(
        _psd_body,
        out_type=jax.ShapeDtypeStruct((_ROWS,), jnp.float32),
        mesh=mesh,
        scratch_types=[
            pltpu.VMEM((_G, _F), jnp.float32),
            pltpu.VMEM((_G, _F), jnp.float32),
            pltpu.VMEM((_RPW,), jnp.float32),
            pltpu.SemaphoreType.DMA,
            pltpu.SemaphoreType.DMA,
        ],
    )(x2d)


def kernel(x):
    rr = _psd_peaks(x.reshape(_ROWS, _F))
    return rr.reshape(_B, _T)


# speedup vs baseline: 1.6369x; 1.1421x over previous
import jax
import jax.numpy as jnp
from jax import lax
from jax.experimental import pallas as pl
from jax.experimental.pallas import tpu as pltpu
from jax.experimental.pallas import tpu_sc as plsc

_FMIN = 0.1
_FMAX = 0.5

_B, _T, _F = 64, 512, 1024
_ROWS = _B * _T
_NW = 32
_L = 16

_TCB = 512
_RT = 44 * _TCB
_RS = _ROWS - _RT
_RPW = _RS // _NW
_G = 32
_STEPS = _RPW // _G


_RPAR = 8


def _rows_argmax(buf, rbase):
    lanes = lax.iota(jnp.int32, _L)

    def chunk(c, carry):
        bests, bestis, ivec = carry
        nb, ni = [], []
        for k in range(_RPAR):
            v = buf[rbase + k, pl.ds(c * _L, _L)]
            m = v > bests[k]
            nb.append(jnp.maximum(bests[k], v))
            ni.append(jnp.where(m, ivec, bestis[k]))
        return tuple(nb), tuple(ni), ivec + _L

    best0 = jnp.full((_L,), -jnp.inf, jnp.float32)
    bests, bestis, _ = lax.fori_loop(
        0, _F // _L, chunk,
        ((best0,) * _RPAR, (lanes,) * _RPAR, lanes),
        unroll=4,
    )
    return bests, bestis


def _perm(x, pidx):
    return lax.gather(
        x,
        pidx[:, None],
        lax.GatherDimensionNumbers(
            offset_dims=(), collapsed_slice_dims=(0,), start_index_map=(0,)
        ),
        slice_sizes=(1,),
        mode=lax.GatherScatterMode.PROMISE_IN_BOUNDS,
    )


def _lane_argmax(best, besti):
    lanes = lax.iota(jnp.int32, _L)
    for shift in (8, 4, 2, 1):
        pidx = lanes ^ shift
        pv = _perm(best, pidx)
        pi = _perm(besti, pidx)
        take = (pv > best) | ((pv == best) & (pi < besti))
        best = jnp.where(take, pv, best)
        besti = jnp.where(take, pi, besti)
    return besti


def _compute_block(buf, out_v, step):
    lanes = lax.iota(jnp.int32, _L)
    scale = (_FMAX - _FMIN) / (_F - 1) * 60.0
    bias = _FMIN * 60.0

    def group(g, _):
        def quad(r4, acc):
            bests, bestis = _rows_argmax(buf, g * _L + r4 * _RPAR)
            for k in range(_RPAR):
                wini = _lane_argmax(bests[k], bestis[k])
                rr = wini.astype(jnp.float32) * scale + bias
                acc = jnp.where(lanes == r4 * _RPAR + k, rr, acc)
            return acc

        acc = lax.fori_loop(0, _L // _RPAR, quad, jnp.zeros((_L,), jnp.float32))
        out_v[pl.ds(step * _G + g * _L, _L)] = acc
        return 0

    lax.fori_loop(0, _G // _L, group, 0)


def _psd_body(x_hbm, out_hbm, buf0, buf1, out_v, sem0, sem1):
    wid = lax.axis_index("s") * 2 + lax.axis_index("c")
    row0 = wid * _RPW
    in0 = _RT + row0

    def start(step, buf, sem):
        pltpu.async_copy(x_hbm.at[pl.ds(in0 + step * _G, _G)], buf, sem)

    def wait(step, buf, sem):
        pltpu.make_async_copy(x_hbm.at[pl.ds(in0 + step * _G, _G)], buf, sem).wait()

    start(0, buf0, sem0)

    def pair(p, _):
        s0 = p * 2
        wait(s0, buf0, sem0)
        start(s0 + 1, buf1, sem1)
        _compute_block(buf0, out_v, s0)
        wait(s0 + 1, buf1, sem1)

        @pl.when(p < _STEPS // 2 - 1)
        def _():
            start(s0 + 2, buf0, sem0)

        _compute_block(buf1, out_v, s0 + 1)
        return 0

    lax.fori_loop(0, _STEPS // 2, pair, 0)
    pltpu.sync_copy(out_v, out_hbm.at[pl.ds(row0, _RPW)])


def _tc_body(x_ref, o_ref):
    x = x_ref[...]
    m = jnp.max(x, axis=1, keepdims=True)
    idx = lax.broadcasted_iota(jnp.int32, x.shape, 1)
    cand = jnp.where(x == m, idx, _F)
    wini = jnp.min(cand, axis=1).astype(jnp.float32)
    scale = (_FMAX - _FMIN) / (_F - 1) * 60.0
    o_ref[0, 0, :] = wini * scale + _FMIN * 60.0


@jax.jit
def _psd_peaks(x2d):
    mesh = plsc.VectorSubcoreMesh(
        core_axis_name="c", subcore_axis_name="s", num_cores=2, num_subcores=16
    )
    sc_rr = pl.kernel(
        _psd_body,
        out_type=jax.ShapeDtypeStruct((_RS,), jnp.float32),
        mesh=mesh,
        scratch_types=[
            pltpu.VMEM((_G, _F), jnp.float32),
            pltpu.VMEM((_G, _F), jnp.float32),
            pltpu.VMEM((_RPW,), jnp.float32),
            pltpu.SemaphoreType.DMA,
            pltpu.SemaphoreType.DMA,
        ],
    )(x2d)
    tc_rr = pl.pallas_call(
        _tc_body,
        grid=(_RT // _TCB,),
        in_specs=[pl.BlockSpec((_TCB, _F), lambda i: (i, 0))],
        out_specs=pl.BlockSpec((1, 1, _TCB), lambda i: (i, 0, 0)),
        out_shape=jax.ShapeDtypeStruct((_RT // _TCB, 1, _TCB), jnp.float32),
    )(x2d)
    return jnp.concatenate([tc_rr.reshape(_RT), sc_rr])


def kernel(x):
    rr = _psd_peaks(x.reshape(_ROWS, _F))
    return rr.reshape(_B, _T)


# speedup vs baseline: 2.8170x; 1.7209x over previous
import jax
import jax.numpy as jnp
from jax import lax
from jax.experimental import pallas as pl
from jax.experimental.pallas import tpu as pltpu

_FMIN, _FMAX = 0.1, 0.5
_B, _T, _F = 64, 512, 1024
_ROWS = _B * _T
_TCB = 512
_NBLK = _ROWS // _TCB
_NBUF = 4


def _tc_block_argmax(buf):
    nch = _F // 128
    best = buf[:, 0:128]
    bestj = jnp.zeros((_TCB, 128), jnp.int32)
    for j in range(1, nch):
        v = buf[:, j * 128:(j + 1) * 128]
        m = v > best
        best = jnp.maximum(best, v)
        bestj = jnp.where(m, j, bestj)
    m2 = jnp.max(best, axis=1, keepdims=True)
    lane = lax.broadcasted_iota(jnp.int32, (_TCB, 128), 1)
    key = bestj * 128 + lane
    cand = jnp.where(best == m2, key, _F)
    wini = jnp.min(cand, axis=1).astype(jnp.float32)
    scale = (_FMAX - _FMIN) / (_F - 1) * 60.0
    return wini * scale + _FMIN * 60.0


def _tc_body(x_hbm, o_hbm, bufs, out_v, sems):
    def start(i, k):
        pltpu.async_copy(x_hbm.at[pl.ds(i * _TCB, _TCB)], bufs.at[k], sems.at[k])

    def wait(i, k):
        pltpu.make_async_copy(
            x_hbm.at[pl.ds(i * _TCB, _TCB)], bufs.at[k], sems.at[k]
        ).wait()

    for k in range(_NBUF):
        start(k, k)

    def ring(p, _):
        for k in range(_NBUF):
            i = p * _NBUF + k
            wait(i, k)
            out_v[i, :] = _tc_block_argmax(bufs.at[k])

            @pl.when(i + _NBUF < _NBLK)
            def _():
                start(i + _NBUF, k)

        return 0

    lax.fori_loop(0, _NBLK // _NBUF, ring, 0)
    pltpu.sync_copy(out_v, o_hbm)


@jax.jit
def _psd_peaks(x2d):
    return pl.pallas_call(
        _tc_body,
        in_specs=[pl.BlockSpec(memory_space=pl.ANY)],
        out_specs=pl.BlockSpec(memory_space=pl.ANY),
        out_shape=jax.ShapeDtypeStruct((_B, _T), jnp.float32),
        scratch_shapes=[
            pltpu.VMEM((_NBUF, _TCB, _F), jnp.float32),
            pltpu.VMEM((_B, _T), jnp.float32),
            pltpu.SemaphoreType.DMA((_NBUF,)),
        ],
    )(x2d)


def kernel(x):
    return _psd_peaks(x.reshape(_ROWS, _F))
